# baseline (speedup 1.0000x reference)
import dataclasses
import functools

import jax
import jax.numpy as jnp
from jax import lax
from jax.experimental import pallas as pl
from jax.experimental.pallas import tpu as pltpu
from jax.experimental.pallas import tpu_sc as plsc

_D = 128
_NF = 400000
_E = 600000
_NS = 16
_L = 16
_C = 11264
_CP = _C + 8
_NCHUNK = 36
_CHUNK_ITERS = 18
_SEG = 1024
_NSEG = 38
_EW = _SEG * _NSEG
_EPAD = _EW * _NS
_STRIPE = _C // _NS
_SENT = 1 << 30


def _compiler_params():
    cp = pltpu.CompilerParams(use_tc_tiling_on_sc=True)
    if "needs_layout_passes" in pltpu.CompilerParams.__dataclass_fields__:
        cp = dataclasses.replace(cp, needs_layout_passes=False)
    return cp


@functools.partial(
    pl.kernel,
    out_type=jax.ShapeDtypeStruct((_NF, _D), jnp.float32),
    mesh=plsc.VectorSubcoreMesh(core_axis_name="c", subcore_axis_name="s"),
    scratch_types=[
        pltpu.VMEM((2, _SEG), jnp.int32),
        pltpu.VMEM((2, _SEG), jnp.int32),
        pltpu.VMEM((768,), jnp.int32),
        pltpu.VMEM((768,), jnp.int32),
        pltpu.VMEM((768,), jnp.int32),
        pltpu.VMEM((768,), jnp.int32),
        pltpu.VMEM((128,), jnp.int32),
        pltpu.VMEM((128,), jnp.int32),
        pltpu.VMEM((128, _D), jnp.float32),
        pltpu.VMEM((128,), jnp.float32),
        pltpu.VMEM((32, _D), jnp.float32),
        pltpu.VMEM((_STRIPE,), jnp.float32),
        pltpu.VMEM((64, _D), jnp.float32),
        pltpu.VMEM((64,), jnp.float32),
        pltpu.VMEM((_L,), jnp.float32),
        pltpu.VMEM_SHARED((_CP, _D), jnp.float32),
        pltpu.VMEM_SHARED((_CP,), jnp.float32),
        pltpu.SemaphoreType.DMA,
        pltpu.SemaphoreType.DMA,
        pltpu.SemaphoreType.DMA,
        pltpu.SemaphoreType.DMA,
        pltpu.SemaphoreType.DMA,
    ],
    compiler_params=_compiler_params(),
)
def _upsample_sc(feat_hbm, src_hbm, tgt_hbm, offc_hbm, out_hbm,
                 tgts_v, srcs_v, sel_src_a, sel_off_a, sel_src_b, sel_off_b,
                 offbuf, srcbuf, rows_v,
                 ones_v, zrow_v, zcnt_v, mean_v, cntl_v, offc_v,
                 sums_sh, cnt_sh, gsem, lsem0, lsem1, msem, zsem):
    cid = lax.axis_index("c")
    sid = lax.axis_index("s")

    base_e = sid * _EW
    pltpu.sync_copy(offc_hbm, offc_v)

    fz = jnp.zeros((_L,), jnp.float32)
    fo = jnp.ones((_L,), jnp.float32)

    @pl.loop(0, 128 // _L)
    def _(q):
        ones_v[pl.ds(q * _L, _L)] = fo

    @pl.loop(0, 32)
    def _(r):
        @pl.loop(0, _D // _L)
        def _(q):
            zrow_v[r, pl.ds(q * _L, _L)] = fz

    @pl.loop(0, _STRIPE // _L)
    def _(q):
        zcnt_v[pl.ds(q * _L, _L)] = fz

    stripe0 = sid * _STRIPE

    @pl.loop(0, _STRIPE // 32)
    def _(b):
        pltpu.sync_copy(zrow_v, sums_sh.at[pl.ds(stripe0 + b * 32, 32)])

    pltpu.sync_copy(zcnt_v, cnt_sh.at[pl.ds(stripe0, _STRIPE)])
    plsc.subcore_barrier()

    @pl.loop(0, _CHUNK_ITERS)
    def _(ci):
        c = ci * 2 + cid

        @pl.when(c < _NCHUNK)
        def _():
            lo = c * _C
            hi = lo + _C

            def mk_flush(ssrc, soff):
                def flush(j, carry):
                    jb = j * 128
                    for q in range(8):
                        srcbuf[pl.ds(q * _L, _L)] = \
                            ssrc[pl.ds(jb + q * _L, _L)]
                    desc = pltpu.async_copy(feat_hbm.at[srcbuf], rows_v,
                                            gsem)
                    for q in range(8):
                        offbuf[pl.ds(q * _L, _L)] = \
                            soff[pl.ds(jb + q * _L, _L)]
                    desc.wait()
                    pltpu.sync_copy(rows_v, sums_sh.at[offbuf], add=True)
                    pltpu.sync_copy(ones_v, cnt_sh.at[offbuf], add=True)
                    return carry
                return flush

            flush_a = mk_flush(sel_src_a, sel_off_a)
            flush_b = mk_flush(sel_src_b, sel_off_b)

            def fire(s, buf, sem):
                seg0 = base_e + s * _SEG
                pltpu.async_copy(tgt_hbm.at[pl.ds(seg0, _SEG)],
                                 tgts_v.at[buf], sem)
                pltpu.async_copy(src_hbm.at[pl.ds(seg0, _SEG)],
                                 srcs_v.at[buf], sem)

            def drain(buf, sem):
                pltpu.make_async_copy(tgt_hbm.at[pl.ds(0, _SEG)],
                                      tgts_v.at[buf], sem).wait()
                pltpu.make_async_copy(src_hbm.at[pl.ds(0, _SEG)],
                                      srcs_v.at[buf], sem).wait()

            def scan_buf(buf, cnt_ab):
                def scan_body(i, c):
                    ca, cb = c
                    pa = i * _L
                    pb = (_SEG // 2) + i * _L
                    ta = tgts_v[buf, pl.ds(pa, _L)]
                    sa = srcs_v[buf, pl.ds(pa, _L)]
                    tb = tgts_v[buf, pl.ds(pb, _L)]
                    sb = srcs_v[buf, pl.ds(pb, _L)]
                    ma = (ta >= lo) & (ta < hi)
                    mb = (tb >= lo) & (tb < hi)
                    plsc.store_compressed(sel_src_a.at[pl.ds(ca, _L)], sa,
                                          mask=ma)
                    plsc.store_compressed(sel_off_a.at[pl.ds(ca, _L)],
                                          ta - lo, mask=ma)
                    plsc.store_compressed(sel_src_b.at[pl.ds(cb, _L)], sb,
                                          mask=mb)
                    plsc.store_compressed(sel_off_b.at[pl.ds(cb, _L)],
                                          tb - lo, mask=mb)
                    return (ca + plsc.all_reduce_population_count(ma)[0],
                            cb + plsc.all_reduce_population_count(mb)[0])

                ca, cb = lax.fori_loop(0, _SEG // (2 * _L), scan_body,
                                       cnt_ab, unroll=8)

                nfa = ca >> 7
                lax.fori_loop(0, nfa, flush_a, jnp.int32(0))
                ra = nfa * 128
                nfb = cb >> 7
                lax.fori_loop(0, nfb, flush_b, jnp.int32(0))
                rb = nfb * 128
                for q in range(8):
                    v0 = sel_src_a[pl.ds(ra + q * _L, _L)]
                    v1 = sel_off_a[pl.ds(ra + q * _L, _L)]
                    v2 = sel_src_b[pl.ds(rb + q * _L, _L)]
                    v3 = sel_off_b[pl.ds(rb + q * _L, _L)]
                    sel_src_a[pl.ds(q * _L, _L)] = v0
                    sel_off_a[pl.ds(q * _L, _L)] = v1
                    sel_src_b[pl.ds(q * _L, _L)] = v2
                    sel_off_b[pl.ds(q * _L, _L)] = v3
                return (ca - ra, cb - rb)

            fire(jnp.int32(0), 0, lsem0)
            fire(jnp.int32(1), 1, lsem1)

            def pair_body(p, cnt):
                s0 = p * 2
                drain(0, lsem0)
                cnt = scan_buf(0, cnt)

                @pl.when(s0 + 2 < _NSEG)
                def _():
                    fire(s0 + 2, 0, lsem0)

                drain(1, lsem1)
                cnt = scan_buf(1, cnt)

                @pl.when(s0 + 3 < _NSEG)
                def _():
                    fire(s0 + 3, 1, lsem1)

                return cnt

            ca_end, cb_end = lax.fori_loop(
                0, _NSEG // 2, pair_body, (jnp.int32(0), jnp.int32(0)))

            def tail_flush(cnt_end, ssrc, soff, fl):
                @pl.when(cnt_end > 0)
                def _():
                    trash = jnp.full((_L,), _C, jnp.int32)
                    mall = jnp.ones((_L,), jnp.bool_)
                    for q in range(8):
                        zsrc = lax.iota(jnp.int32, _L) + (q * _L)
                        plsc.store_compressed(
                            soff.at[pl.ds(cnt_end + q * _L, _L)], trash,
                            mask=mall)
                        plsc.store_compressed(
                            ssrc.at[pl.ds(cnt_end + q * _L, _L)], zsrc,
                            mask=mall)
                    fl(jnp.int32(0), jnp.int32(0))

            tail_flush(ca_end, sel_src_a, sel_off_a, flush_a)
            tail_flush(cb_end, sel_src_b, sel_off_b, flush_b)

            plsc.subcore_barrier()

            offv = offc_v[...]

            @pl.loop(0, _STRIPE // 64)
            def _(b):
                r0 = stripe0 + b * 64
                grow = lo + r0

                @pl.when(grow < _NF)
                def _():
                    pltpu.async_copy(sums_sh.at[pl.ds(r0, 64)], mean_v, msem)
                    pltpu.async_copy(cnt_sh.at[pl.ds(r0, 64)], cntl_v, msem)
                    pltpu.make_async_copy(
                        sums_sh.at[pl.ds(r0, 64)], mean_v, msem).wait()
                    pltpu.make_async_copy(
                        cnt_sh.at[pl.ds(r0, 64)], cntl_v, msem).wait()

                    for h in range(4):
                        cv = cntl_v[pl.ds(h * _L, _L)]
                        iv = 1.0 / jnp.maximum(cv, 1.0)
                        for r in range(_L):
                            row = h * _L + r
                            cinv = iv[r]

                            @pl.loop(0, _D // _L)
                            def _(q, row=row, cinv=cinv):
                                v = mean_v[row, pl.ds(q * _L, _L)]
                                mean_v[row, pl.ds(q * _L, _L)] = \
                                    v * cinv + offv

                    pltpu.sync_copy(mean_v, out_hbm.at[pl.ds(grow, 64)])

                pltpu.async_copy(zrow_v, sums_sh.at[pl.ds(r0, 32)], zsem)
                pltpu.async_copy(zrow_v, sums_sh.at[pl.ds(r0 + 32, 32)],
                                 zsem)

            pltpu.sync_copy(zcnt_v, cnt_sh.at[pl.ds(stripe0, _STRIPE)])

            @pl.loop(0, _STRIPE // 64)
            def _(b):
                r0 = stripe0 + b * 64
                pltpu.make_async_copy(zrow_v, sums_sh.at[pl.ds(r0, 32)],
                                      zsem).wait()
                pltpu.make_async_copy(zrow_v, sums_sh.at[pl.ds(r0 + 32, 32)],
                                      zsem).wait()
            plsc.subcore_barrier()


def kernel(feat, src_ids, tgt_ids, dim_size, feat_depth):
    src_p = jnp.concatenate(
        [src_ids.astype(jnp.int32), jnp.zeros((_EPAD - _E,), jnp.int32)])
    tgt_p = jnp.concatenate(
        [tgt_ids.astype(jnp.int32), jnp.full((_EPAD - _E,), _SENT, jnp.int32)])
    offc = jnp.full((_L,), jnp.asarray(dim_size, jnp.float32) - float(_NF))
    out = _upsample_sc(feat, src_p, tgt_p, offc)
    return (out, feat_depth - 1)


# speedup vs baseline: 1.0473x; 1.0473x over previous
import dataclasses
import functools

import jax
import jax.numpy as jnp
from jax import lax
from jax.experimental import pallas as pl
from jax.experimental.pallas import tpu as pltpu
from jax.experimental.pallas import tpu_sc as plsc

_D = 128
_NF = 400000
_E = 600000
_NS = 16
_L = 16
_C = 11264
_CP = _C + 8
_NCHUNK = 36
_CHUNK_ITERS = 18
_SEG = 1024
_NSEG = 38
_EW = _SEG * _NSEG
_EPAD = _EW * _NS
_STRIPE = _C // _NS
_SENT = 1 << 30


def _compiler_params():
    cp = pltpu.CompilerParams(use_tc_tiling_on_sc=True)
    if "needs_layout_passes" in pltpu.CompilerParams.__dataclass_fields__:
        cp = dataclasses.replace(cp, needs_layout_passes=False)
    return cp


@functools.partial(
    pl.kernel,
    out_type=jax.ShapeDtypeStruct((_NF, _D), jnp.float32),
    mesh=plsc.VectorSubcoreMesh(core_axis_name="c", subcore_axis_name="s"),
    scratch_types=[
        pltpu.VMEM((2, _SEG), jnp.int32),
        pltpu.VMEM((2, _SEG), jnp.int32),
        pltpu.VMEM((768,), jnp.int32),
        pltpu.VMEM((768,), jnp.int32),
        pltpu.VMEM((768,), jnp.int32),
        pltpu.VMEM((768,), jnp.int32),
        pltpu.VMEM((2, 64), jnp.int32),
        pltpu.VMEM((128,), jnp.int32),
        pltpu.VMEM((128, _D), jnp.float32),
        pltpu.VMEM((128,), jnp.float32),
        pltpu.VMEM((32, _D), jnp.float32),
        pltpu.VMEM((_STRIPE,), jnp.float32),
        pltpu.VMEM((64, _D), jnp.float32),
        pltpu.VMEM((64,), jnp.float32),
        pltpu.VMEM((_L,), jnp.float32),
        pltpu.VMEM_SHARED((_CP, _D), jnp.float32),
        pltpu.VMEM_SHARED((_CP,), jnp.float32),
        pltpu.SemaphoreType.DMA,
        pltpu.SemaphoreType.DMA,
        pltpu.SemaphoreType.DMA,
        pltpu.SemaphoreType.DMA,
        pltpu.SemaphoreType.DMA,
        pltpu.SemaphoreType.DMA,
        pltpu.SemaphoreType.DMA,
    ],
    compiler_params=_compiler_params(),
)
def _upsample_sc(feat_hbm, src_hbm, tgt_hbm, offc_hbm, out_hbm,
                 tgts_v, srcs_v, sel_src_a, sel_off_a, sel_src_b, sel_off_b,
                 offbuf, srcbuf, rows_v,
                 ones_v, zrow_v, zcnt_v, mean_v, cntl_v, offc_v,
                 sums_sh, cnt_sh, gsem, lsem0, lsem1, msem, zsem,
                 ssem, csem):
    cid = lax.axis_index("c")
    sid = lax.axis_index("s")

    base_e = sid * _EW
    pltpu.sync_copy(offc_hbm, offc_v)

    fz = jnp.zeros((_L,), jnp.float32)
    fo = jnp.ones((_L,), jnp.float32)

    @pl.loop(0, 128 // _L)
    def _(q):
        ones_v[pl.ds(q * _L, _L)] = fo

    @pl.loop(0, 32)
    def _(r):
        @pl.loop(0, _D // _L)
        def _(q):
            zrow_v[r, pl.ds(q * _L, _L)] = fz

    @pl.loop(0, _STRIPE // _L)
    def _(q):
        zcnt_v[pl.ds(q * _L, _L)] = fz

    stripe0 = sid * _STRIPE

    @pl.loop(0, _STRIPE // 32)
    def _(b):
        pltpu.sync_copy(zrow_v, sums_sh.at[pl.ds(stripe0 + b * 32, 32)])

    pltpu.sync_copy(zcnt_v, cnt_sh.at[pl.ds(stripe0, _STRIPE)])
    plsc.subcore_barrier()

    @pl.loop(0, _CHUNK_ITERS)
    def _(ci):
        c = ci * 2 + cid

        @pl.when(c < _NCHUNK)
        def _():
            lo = c * _C
            hi = lo + _C

            def mk_flush(ssrc, soff):
                def flush(j, carry):
                    jb = j * 128
                    for q in range(4):
                        srcbuf[pl.ds(q * _L, _L)] = \
                            ssrc[pl.ds(jb + q * _L, _L)]
                    dlo = pltpu.async_copy(
                        feat_hbm.at[srcbuf.at[pl.ds(0, 64)]],
                        rows_v.at[pl.ds(0, 64)], gsem)
                    for q in range(4):
                        srcbuf[pl.ds(64 + q * _L, _L)] = \
                            ssrc[pl.ds(jb + 64 + q * _L, _L)]
                    dhi = pltpu.async_copy(
                        feat_hbm.at[srcbuf.at[pl.ds(64, 64)]],
                        rows_v.at[pl.ds(64, 64)], gsem)
                    for h in range(2):
                        for q in range(4):
                            offbuf[h, pl.ds(q * _L, _L)] = \
                                soff[pl.ds(jb + h * 64 + q * _L, _L)]
                    dlo.wait()
                    slo = pltpu.async_copy(
                        rows_v.at[pl.ds(0, 64)],
                        sums_sh.at[offbuf.at[0]], ssem, add=True)
                    clo = pltpu.async_copy(
                        ones_v.at[pl.ds(0, 64)],
                        cnt_sh.at[offbuf.at[0]], csem, add=True)
                    dhi.wait()
                    shi = pltpu.async_copy(
                        rows_v.at[pl.ds(64, 64)],
                        sums_sh.at[offbuf.at[1]], ssem, add=True)
                    chi = pltpu.async_copy(
                        ones_v.at[pl.ds(64, 64)],
                        cnt_sh.at[offbuf.at[1]], csem, add=True)
                    slo.wait()
                    shi.wait()
                    clo.wait()
                    chi.wait()
                    return carry
                return flush

            flush_a = mk_flush(sel_src_a, sel_off_a)
            flush_b = mk_flush(sel_src_b, sel_off_b)

            def fire(s, buf, sem):
                seg0 = base_e + s * _SEG
                pltpu.async_copy(tgt_hbm.at[pl.ds(seg0, _SEG)],
                                 tgts_v.at[buf], sem)
                pltpu.async_copy(src_hbm.at[pl.ds(seg0, _SEG)],
                                 srcs_v.at[buf], sem)

            def drain(buf, sem):
                pltpu.make_async_copy(tgt_hbm.at[pl.ds(0, _SEG)],
                                      tgts_v.at[buf], sem).wait()
                pltpu.make_async_copy(src_hbm.at[pl.ds(0, _SEG)],
                                      srcs_v.at[buf], sem).wait()

            def scan_buf(buf, cnt_ab):
                def scan_body(i, c):
                    ca, cb = c
                    pa = i * _L
                    pb = (_SEG // 2) + i * _L
                    ta = tgts_v[buf, pl.ds(pa, _L)]
                    sa = srcs_v[buf, pl.ds(pa, _L)]
                    tb = tgts_v[buf, pl.ds(pb, _L)]
                    sb = srcs_v[buf, pl.ds(pb, _L)]
                    ma = (ta >= lo) & (ta < hi)
                    mb = (tb >= lo) & (tb < hi)
                    plsc.store_compressed(sel_src_a.at[pl.ds(ca, _L)], sa,
                                          mask=ma)
                    plsc.store_compressed(sel_off_a.at[pl.ds(ca, _L)],
                                          ta - lo, mask=ma)
                    plsc.store_compressed(sel_src_b.at[pl.ds(cb, _L)], sb,
                                          mask=mb)
                    plsc.store_compressed(sel_off_b.at[pl.ds(cb, _L)],
                                          tb - lo, mask=mb)
                    return (ca + plsc.all_reduce_population_count(ma)[0],
                            cb + plsc.all_reduce_population_count(mb)[0])

                ca, cb = lax.fori_loop(0, _SEG // (2 * _L), scan_body,
                                       cnt_ab, unroll=4)

                nfa = ca >> 7
                lax.fori_loop(0, nfa, flush_a, jnp.int32(0))
                ra = nfa * 128
                nfb = cb >> 7
                lax.fori_loop(0, nfb, flush_b, jnp.int32(0))
                rb = nfb * 128
                for q in range(8):
                    v0 = sel_src_a[pl.ds(ra + q * _L, _L)]
                    v1 = sel_off_a[pl.ds(ra + q * _L, _L)]
                    v2 = sel_src_b[pl.ds(rb + q * _L, _L)]
                    v3 = sel_off_b[pl.ds(rb + q * _L, _L)]
                    sel_src_a[pl.ds(q * _L, _L)] = v0
                    sel_off_a[pl.ds(q * _L, _L)] = v1
                    sel_src_b[pl.ds(q * _L, _L)] = v2
                    sel_off_b[pl.ds(q * _L, _L)] = v3
                return (ca - ra, cb - rb)

            fire(jnp.int32(0), 0, lsem0)
            fire(jnp.int32(1), 1, lsem1)

            def pair_body(p, cnt):
                s0 = p * 2
                drain(0, lsem0)
                cnt = scan_buf(0, cnt)

                @pl.when(s0 + 2 < _NSEG)
                def _():
                    fire(s0 + 2, 0, lsem0)

                drain(1, lsem1)
                cnt = scan_buf(1, cnt)

                @pl.when(s0 + 3 < _NSEG)
                def _():
                    fire(s0 + 3, 1, lsem1)

                return cnt

            ca_end, cb_end = lax.fori_loop(
                0, _NSEG // 2, pair_body, (jnp.int32(0), jnp.int32(0)))

            def tail_flush(cnt_end, ssrc, soff, fl):
                @pl.when(cnt_end > 0)
                def _():
                    trash = jnp.full((_L,), _C, jnp.int32)
                    mall = jnp.ones((_L,), jnp.bool_)
                    for q in range(8):
                        zsrc = lax.iota(jnp.int32, _L) + (q * _L)
                        plsc.store_compressed(
                            soff.at[pl.ds(cnt_end + q * _L, _L)], trash,
                            mask=mall)
                        plsc.store_compressed(
                            ssrc.at[pl.ds(cnt_end + q * _L, _L)], zsrc,
                            mask=mall)
                    fl(jnp.int32(0), jnp.int32(0))

            tail_flush(ca_end, sel_src_a, sel_off_a, flush_a)
            tail_flush(cb_end, sel_src_b, sel_off_b, flush_b)

            plsc.subcore_barrier()

            offv = offc_v[...]

            @pl.loop(0, _STRIPE // 64)
            def _(b):
                r0 = stripe0 + b * 64
                grow = lo + r0

                @pl.when(grow < _NF)
                def _():
                    pltpu.async_copy(sums_sh.at[pl.ds(r0, 64)], mean_v, msem)
                    pltpu.async_copy(cnt_sh.at[pl.ds(r0, 64)], cntl_v, msem)
                    pltpu.make_async_copy(
                        sums_sh.at[pl.ds(r0, 64)], mean_v, msem).wait()
                    pltpu.make_async_copy(
                        cnt_sh.at[pl.ds(r0, 64)], cntl_v, msem).wait()

                    for h in range(4):
                        cv = cntl_v[pl.ds(h * _L, _L)]
                        iv = 1.0 / jnp.maximum(cv, 1.0)
                        for r in range(_L):
                            row = h * _L + r
                            cinv = iv[r]

                            @pl.loop(0, _D // _L)
                            def _(q, row=row, cinv=cinv):
                                v = mean_v[row, pl.ds(q * _L, _L)]
                                mean_v[row, pl.ds(q * _L, _L)] = \
                                    v * cinv + offv

                    pltpu.sync_copy(mean_v, out_hbm.at[pl.ds(grow, 64)])

                pltpu.async_copy(zrow_v, sums_sh.at[pl.ds(r0, 32)], zsem)
                pltpu.async_copy(zrow_v, sums_sh.at[pl.ds(r0 + 32, 32)],
                                 zsem)

            pltpu.sync_copy(zcnt_v, cnt_sh.at[pl.ds(stripe0, _STRIPE)])

            @pl.loop(0, _STRIPE // 64)
            def _(b):
                r0 = stripe0 + b * 64
                pltpu.make_async_copy(zrow_v, sums_sh.at[pl.ds(r0, 32)],
                                      zsem).wait()
                pltpu.make_async_copy(zrow_v, sums_sh.at[pl.ds(r0 + 32, 32)],
                                      zsem).wait()
            plsc.subcore_barrier()


def kernel(feat, src_ids, tgt_ids, dim_size, feat_depth):
    src_p = jnp.concatenate(
        [src_ids.astype(jnp.int32), jnp.zeros((_EPAD - _E,), jnp.int32)])
    tgt_p = jnp.concatenate(
        [tgt_ids.astype(jnp.int32), jnp.full((_EPAD - _E,), _SENT, jnp.int32)])
    offc = jnp.full((_L,), jnp.asarray(dim_size, jnp.float32) - float(_NF))
    out = _upsample_sc(feat, src_p, tgt_p, offc)
    return (out, feat_depth - 1)


# speedup vs baseline: 1.1373x; 1.0860x over previous
import dataclasses
import functools

import jax
import jax.numpy as jnp
from jax import lax
from jax.experimental import pallas as pl
from jax.experimental.pallas import tpu as pltpu
from jax.experimental.pallas import tpu_sc as plsc

_D = 128
_NF = 400000
_E = 600000
_NS = 16
_L = 16
_C = 11264
_CP = _C + 8
_NCHUNK = 36
_CHUNK_ITERS = 18
_SEG = 1024
_NSEG = 38
_EW = _SEG * _NSEG
_EPAD = _EW * _NS
_STRIPE = _C // _NS
_SENT = 1 << 30


def _compiler_params():
    cp = pltpu.CompilerParams(use_tc_tiling_on_sc=True)
    if "needs_layout_passes" in pltpu.CompilerParams.__dataclass_fields__:
        cp = dataclasses.replace(cp, needs_layout_passes=False)
    return cp


@functools.partial(
    pl.kernel,
    out_type=jax.ShapeDtypeStruct((_NF, _D), jnp.float32),
    mesh=plsc.VectorSubcoreMesh(core_axis_name="c", subcore_axis_name="s"),
    scratch_types=[
        pltpu.VMEM((2, _SEG), jnp.int32),
        pltpu.VMEM((2, _SEG), jnp.int32),
        pltpu.VMEM((768,), jnp.int32),
        pltpu.VMEM((768,), jnp.int32),
        pltpu.VMEM((768,), jnp.int32),
        pltpu.VMEM((768,), jnp.int32),
        pltpu.VMEM((2, 64), jnp.int32),
        pltpu.VMEM((128,), jnp.int32),
        pltpu.VMEM((128, _D), jnp.float32),
        pltpu.VMEM((128,), jnp.float32),
        pltpu.VMEM((32, _D), jnp.float32),
        pltpu.VMEM((_STRIPE,), jnp.float32),
        pltpu.VMEM((2, 32, _D), jnp.float32),
        pltpu.VMEM((2, 32), jnp.float32),
        pltpu.VMEM((_L,), jnp.float32),
        pltpu.VMEM_SHARED((_CP, _D), jnp.float32),
        pltpu.VMEM_SHARED((_CP,), jnp.float32),
        pltpu.SemaphoreType.DMA,
        pltpu.SemaphoreType.DMA,
        pltpu.SemaphoreType.DMA,
        pltpu.SemaphoreType.DMA,
        pltpu.SemaphoreType.DMA,
        pltpu.SemaphoreType.DMA,
        pltpu.SemaphoreType.DMA,
        pltpu.SemaphoreType.DMA,
        pltpu.SemaphoreType.DMA,
        pltpu.SemaphoreType.DMA,
    ],
    compiler_params=_compiler_params(),
)
def _upsample_sc(feat_hbm, src_hbm, tgt_hbm, offc_hbm, out_hbm,
                 tgts_v, srcs_v, sel_src_a, sel_off_a, sel_src_b, sel_off_b,
                 offbuf, srcbuf, rows_v,
                 ones_v, zrow_v, zcnt_v, mean_v, cntl_v, offc_v,
                 sums_sh, cnt_sh, gsem, lsem0, lsem1, msem, zsem,
                 ssem, csem, ml1, mo0, mo1):
    cid = lax.axis_index("c")
    sid = lax.axis_index("s")

    base_e = sid * _EW
    pltpu.sync_copy(offc_hbm, offc_v)

    fz = jnp.zeros((_L,), jnp.float32)
    fo = jnp.ones((_L,), jnp.float32)

    @pl.loop(0, 128 // _L)
    def _(q):
        ones_v[pl.ds(q * _L, _L)] = fo

    @pl.loop(0, 32)
    def _(r):
        @pl.loop(0, _D // _L)
        def _(q):
            zrow_v[r, pl.ds(q * _L, _L)] = fz

    @pl.loop(0, _STRIPE // _L)
    def _(q):
        zcnt_v[pl.ds(q * _L, _L)] = fz

    stripe0 = sid * _STRIPE

    @pl.loop(0, _STRIPE // 32)
    def _(b):
        pltpu.sync_copy(zrow_v, sums_sh.at[pl.ds(stripe0 + b * 32, 32)])

    pltpu.sync_copy(zcnt_v, cnt_sh.at[pl.ds(stripe0, _STRIPE)])
    plsc.subcore_barrier()

    @pl.loop(0, _CHUNK_ITERS)
    def _(ci):
        c = ci * 2 + cid

        @pl.when(c < _NCHUNK)
        def _():
            lo = c * _C
            hi = lo + _C

            def mk_flush(ssrc, soff):
                def flush(j, carry):
                    jb = j * 128
                    for q in range(4):
                        srcbuf[pl.ds(q * _L, _L)] = \
                            ssrc[pl.ds(jb + q * _L, _L)]
                    dlo = pltpu.async_copy(
                        feat_hbm.at[srcbuf.at[pl.ds(0, 64)]],
                        rows_v.at[pl.ds(0, 64)], gsem)
                    for q in range(4):
                        srcbuf[pl.ds(64 + q * _L, _L)] = \
                            ssrc[pl.ds(jb + 64 + q * _L, _L)]
                    dhi = pltpu.async_copy(
                        feat_hbm.at[srcbuf.at[pl.ds(64, 64)]],
                        rows_v.at[pl.ds(64, 64)], gsem)
                    for h in range(2):
                        for q in range(4):
                            offbuf[h, pl.ds(q * _L, _L)] = \
                                soff[pl.ds(jb + h * 64 + q * _L, _L)]
                    dlo.wait()
                    slo = pltpu.async_copy(
                        rows_v.at[pl.ds(0, 64)],
                        sums_sh.at[offbuf.at[0]], ssem, add=True)
                    clo = pltpu.async_copy(
                        ones_v.at[pl.ds(0, 64)],
                        cnt_sh.at[offbuf.at[0]], csem, add=True)
                    dhi.wait()
                    shi = pltpu.async_copy(
                        rows_v.at[pl.ds(64, 64)],
                        sums_sh.at[offbuf.at[1]], ssem, add=True)
                    chi = pltpu.async_copy(
                        ones_v.at[pl.ds(64, 64)],
                        cnt_sh.at[offbuf.at[1]], csem, add=True)
                    slo.wait()
                    shi.wait()
                    clo.wait()
                    chi.wait()
                    return carry
                return flush

            flush_a = mk_flush(sel_src_a, sel_off_a)
            flush_b = mk_flush(sel_src_b, sel_off_b)

            def fire(s, buf, sem):
                seg0 = base_e + s * _SEG
                pltpu.async_copy(tgt_hbm.at[pl.ds(seg0, _SEG)],
                                 tgts_v.at[buf], sem)
                pltpu.async_copy(src_hbm.at[pl.ds(seg0, _SEG)],
                                 srcs_v.at[buf], sem)

            def drain(buf, sem):
                pltpu.make_async_copy(tgt_hbm.at[pl.ds(0, _SEG)],
                                      tgts_v.at[buf], sem).wait()
                pltpu.make_async_copy(src_hbm.at[pl.ds(0, _SEG)],
                                      srcs_v.at[buf], sem).wait()

            def scan_buf(buf, cnt_ab):
                def scan_body(i, c):
                    ca, cb = c
                    pa = i * _L
                    pb = (_SEG // 2) + i * _L
                    ta = tgts_v[buf, pl.ds(pa, _L)]
                    sa = srcs_v[buf, pl.ds(pa, _L)]
                    tb = tgts_v[buf, pl.ds(pb, _L)]
                    sb = srcs_v[buf, pl.ds(pb, _L)]
                    ma = (ta >= lo) & (ta < hi)
                    mb = (tb >= lo) & (tb < hi)
                    plsc.store_compressed(sel_src_a.at[pl.ds(ca, _L)], sa,
                                          mask=ma)
                    plsc.store_compressed(sel_off_a.at[pl.ds(ca, _L)],
                                          ta - lo, mask=ma)
                    plsc.store_compressed(sel_src_b.at[pl.ds(cb, _L)], sb,
                                          mask=mb)
                    plsc.store_compressed(sel_off_b.at[pl.ds(cb, _L)],
                                          tb - lo, mask=mb)
                    return (ca + plsc.all_reduce_population_count(ma)[0],
                            cb + plsc.all_reduce_population_count(mb)[0])

                ca, cb = lax.fori_loop(0, _SEG // (2 * _L), scan_body,
                                       cnt_ab, unroll=4)

                nfa = ca >> 7
                lax.fori_loop(0, nfa, flush_a, jnp.int32(0))
                ra = nfa * 128
                nfb = cb >> 7
                lax.fori_loop(0, nfb, flush_b, jnp.int32(0))
                rb = nfb * 128
                for q in range(8):
                    v0 = sel_src_a[pl.ds(ra + q * _L, _L)]
                    v1 = sel_off_a[pl.ds(ra + q * _L, _L)]
                    v2 = sel_src_b[pl.ds(rb + q * _L, _L)]
                    v3 = sel_off_b[pl.ds(rb + q * _L, _L)]
                    sel_src_a[pl.ds(q * _L, _L)] = v0
                    sel_off_a[pl.ds(q * _L, _L)] = v1
                    sel_src_b[pl.ds(q * _L, _L)] = v2
                    sel_off_b[pl.ds(q * _L, _L)] = v3
                return (ca - ra, cb - rb)

            fire(jnp.int32(0), 0, lsem0)
            fire(jnp.int32(1), 1, lsem1)

            def pair_body(p, cnt):
                s0 = p * 2
                drain(0, lsem0)
                cnt = scan_buf(0, cnt)

                @pl.when(s0 + 2 < _NSEG)
                def _():
                    fire(s0 + 2, 0, lsem0)

                drain(1, lsem1)
                cnt = scan_buf(1, cnt)

                @pl.when(s0 + 3 < _NSEG)
                def _():
                    fire(s0 + 3, 1, lsem1)

                return cnt

            ca_end, cb_end = lax.fori_loop(
                0, _NSEG // 2, pair_body, (jnp.int32(0), jnp.int32(0)))

            def tail_flush(cnt_end, ssrc, soff, fl):
                @pl.when(cnt_end > 0)
                def _():
                    trash = jnp.full((_L,), _C, jnp.int32)
                    mall = jnp.ones((_L,), jnp.bool_)
                    for q in range(8):
                        zsrc = lax.iota(jnp.int32, _L) + (q * _L)
                        plsc.store_compressed(
                            soff.at[pl.ds(cnt_end + q * _L, _L)], trash,
                            mask=mall)
                        plsc.store_compressed(
                            ssrc.at[pl.ds(cnt_end + q * _L, _L)], zsrc,
                            mask=mall)
                    fl(jnp.int32(0), jnp.int32(0))

            tail_flush(ca_end, sel_src_a, sel_off_a, flush_a)
            tail_flush(cb_end, sel_src_b, sel_off_b, flush_b)

            plsc.subcore_barrier()

            offv = offc_v[...]

            def m_load(b, mb, sem):
                r0 = stripe0 + b * 32

                @pl.when(lo + r0 < _NF)
                def _():
                    pltpu.async_copy(sums_sh.at[pl.ds(r0, 32)],
                                     mean_v.at[mb], sem)
                    pltpu.async_copy(cnt_sh.at[pl.ds(r0, 32)],
                                     cntl_v.at[mb], sem)

            def m_wait_load(b, mb, sem):
                r0 = stripe0 + b * 32

                @pl.when(lo + r0 < _NF)
                def _():
                    pltpu.make_async_copy(sums_sh.at[pl.ds(r0, 32)],
                                          mean_v.at[mb], sem).wait()
                    pltpu.make_async_copy(cnt_sh.at[pl.ds(r0, 32)],
                                          cntl_v.at[mb], sem).wait()

                pltpu.async_copy(zrow_v, sums_sh.at[pl.ds(r0, 32)], zsem)

            def m_compute_store(b, mb, osem):
                r0 = stripe0 + b * 32
                grow = lo + r0

                @pl.when(grow < _NF)
                def _():
                    for h in range(2):
                        cv = cntl_v[mb, pl.ds(h * _L, _L)]
                        iv = 1.0 / jnp.maximum(cv, 1.0)
                        for r in range(_L):
                            row = h * _L + r
                            cinv = iv[r]

                            @pl.loop(0, _D // _L)
                            def _(q, row=row, cinv=cinv):
                                v = mean_v[mb, row, pl.ds(q * _L, _L)]
                                mean_v[mb, row, pl.ds(q * _L, _L)] = \
                                    v * cinv + offv

                    pltpu.async_copy(mean_v.at[mb],
                                     out_hbm.at[pl.ds(grow, 32)], osem)

            def m_wait_store(b, mb, osem):
                r0 = stripe0 + b * 32
                grow = lo + r0

                @pl.when(grow < _NF)
                def _():
                    pltpu.make_async_copy(mean_v.at[mb],
                                          out_hbm.at[pl.ds(grow, 32)],
                                          osem).wait()

            npair = _STRIPE // 64
            m_load(0, 0, msem)

            @pl.loop(0, npair)
            def _(p):
                b0 = 2 * p
                b1 = b0 + 1

                @pl.when(p > 0)
                def _():
                    m_wait_store(b1 - 2, 1, mo1)

                m_load(b1, 1, ml1)
                m_wait_load(b0, 0, msem)
                m_compute_store(b0, 0, mo0)
                m_wait_load(b1, 1, ml1)
                m_compute_store(b1, 1, mo1)

                @pl.when(p < npair - 1)
                def _():
                    m_wait_store(b0, 0, mo0)
                    m_load(b0 + 2, 0, msem)

            m_wait_store(_STRIPE // 32 - 2, 0, mo0)
            m_wait_store(_STRIPE // 32 - 1, 1, mo1)

            pltpu.sync_copy(zcnt_v, cnt_sh.at[pl.ds(stripe0, _STRIPE)])

            @pl.loop(0, _STRIPE // 32)
            def _(b):
                r0 = stripe0 + b * 32
                pltpu.make_async_copy(zrow_v, sums_sh.at[pl.ds(r0, 32)],
                                      zsem).wait()
            plsc.subcore_barrier()


def kernel(feat, src_ids, tgt_ids, dim_size, feat_depth):
    src_p = jnp.concatenate(
        [src_ids.astype(jnp.int32), jnp.zeros((_EPAD - _E,), jnp.int32)])
    tgt_p = jnp.concatenate(
        [tgt_ids.astype(jnp.int32), jnp.full((_EPAD - _E,), _SENT, jnp.int32)])
    offc = jnp.full((_L,), jnp.asarray(dim_size, jnp.float32) - float(_NF))
    out = _upsample_sc(feat, src_p, tgt_p, offc)
    return (out, feat_depth - 1)
